# baseline (speedup 1.0000x reference)
import functools
import jax
import jax.numpy as jnp
import numpy as np
from jax import lax
from jax.experimental import pallas as pl
from jax.experimental.pallas import tpu as pltpu
from jax.experimental.pallas import tpu_sc as plsc

D_K = 1024
S_K = 8192
B_K = 4
SCALE_K = float(np.sqrt(D_K))


SC_B = 2
NC_K, NS_K, L_K = 2, 16, 16
NW_K = NC_K * NS_K
ROWS_W = S_K // NW_K
R_K = 16
CHUNKS_K = ROWS_W // R_K
GROUPS_K = R_K * D_K // L_K
CPR_K = D_K // L_K
NB_K = 5
LOOK_K = 3
NT_K = CHUNKS_K * SC_B

_mesh = plsc.VectorSubcoreMesh(core_axis_name="c", subcore_axis_name="s")


@functools.partial(
    pl.kernel,
    mesh=_mesh,
    out_type=jax.ShapeDtypeStruct((SC_B * S_K, D_K), jnp.float32),
    scratch_types=(
        [pltpu.VMEM((R_K, D_K), jnp.float32) for _ in range(NB_K + 2)]
        + [pltpu.SemaphoreType.DMA for _ in range(2 * NB_K + 2)]
    ),
)
def _sc_add(x_hbm, pe_hbm, out_hbm, xb0, xb1, xb2, xb3, xb4, pb0, pb1,
            si0, si1, si2, si3, si4, so0, so1, so2, so3, so4, sp0, sp1):
    xb = [xb0, xb1, xb2, xb3, xb4]
    si = [si0, si1, si2, si3, si4]
    so = [so0, so1, so2, so3, so4]
    pb = [pb0, pb1]
    sp = [sp0, sp1]

    wid = lax.axis_index("s") * NC_K + lax.axis_index("c")
    base = wid * ROWS_W

    def pe_row(c):
        return pl.multiple_of(base + c * R_K, R_K)

    def x_row(t):
        c, b = divmod(t, SC_B)
        return pl.multiple_of(b * S_K + base + c * R_K, R_K)

    def in_copy(t):
        return pltpu.make_async_copy(
            x_hbm.at[pl.ds(x_row(t), R_K)], xb[t % NB_K], si[t % NB_K])

    def out_copy(t):
        return pltpu.make_async_copy(
            xb[t % NB_K], out_hbm.at[pl.ds(x_row(t), R_K)], so[t % NB_K])

    def pe_copy(c):
        return pltpu.make_async_copy(
            pe_hbm.at[pl.ds(pe_row(c), R_K)], pb[c % 2], sp[c % 2])

    pe_copy(0).start()
    pe_copy(1).start()
    for t in range(LOOK_K):
        in_copy(t).start()

    for t in range(NT_K):
        nxt = t + LOOK_K
        if nxt < NT_K:
            if nxt >= NB_K:
                out_copy(nxt - NB_K).wait()
            in_copy(nxt).start()

        c, b = divmod(t, SC_B)
        if b == 0:
            pe_copy(c).wait()

        in_copy(t).wait()
        xbt = xb[t % NB_K]
        pebt = pb[c % 2]

        @plsc.parallel_loop(0, GROUPS_K, 1, unroll=16)
        def _add(g):
            r = g >> 6
            col = (g & (CPR_K - 1)) * L_K
            sl = pl.ds(col, L_K)
            xbt[r, sl] = xbt[r, sl] + pebt[r, sl] * SCALE_K

        out_copy(t).start()

        if b == SC_B - 1 and c + 2 < CHUNKS_K:
            pe_copy(c + 2).start()

    for t in range(NT_K - NB_K, NT_K):
        out_copy(t).wait()


TC_B = B_K - SC_B
SEQ_BLK = 256


def _tc_body(x_ref, pe_ref, o_ref):
    o_ref[...] = x_ref[...] + pe_ref[...][None] * SCALE_K


def _tc_add(x, pe):
    return pl.pallas_call(
        _tc_body,
        grid=(S_K // SEQ_BLK, TC_B),
        in_specs=[
            pl.BlockSpec((1, SEQ_BLK, D_K), lambda i, j: (SC_B + j, i, 0)),
            pl.BlockSpec((SEQ_BLK, D_K), lambda i, j: (i, 0)),
        ],
        out_specs=pl.BlockSpec((1, SEQ_BLK, D_K), lambda i, j: (j, i, 0)),
        out_shape=jax.ShapeDtypeStruct((TC_B, S_K, D_K), jnp.float32),
    )(x, pe)


def kernel(x, pe_weight):
    b, s, d = x.shape
    sc_out = _sc_add(x.reshape(b * s, d), pe_weight)
    tc_out = _tc_add(x, pe_weight)
    return jnp.concatenate([sc_out.reshape(SC_B, s, d), tc_out], axis=0)


# speedup vs baseline: 1.6368x; 1.6368x over previous
import functools
import jax
import jax.numpy as jnp
import numpy as np
from jax import lax
from jax.experimental import pallas as pl
from jax.experimental.pallas import tpu as pltpu
from jax.experimental.pallas import tpu_sc as plsc

D_K = 1024
S_K = 8192
B_K = 4
SCALE_K = float(np.sqrt(D_K))
NC_K, NS_K, L_K = 2, 16, 16
NW_K = NC_K * NS_K
ROWS_W = S_K // NW_K
R_K = 16
H_K = R_K // 2
CHUNKS_K = ROWS_W // R_K
GROUPS_K = R_K * D_K // L_K
CPR_K = D_K // L_K
NB_K = 5
LOOK_K = 3
NSLOT = 2
NDO = 3
NT_K = CHUNKS_K * B_K

_mesh = plsc.VectorSubcoreMesh(core_axis_name="c", subcore_axis_name="s")


@functools.partial(
    pl.kernel,
    mesh=_mesh,
    out_type=jax.ShapeDtypeStruct((B_K * S_K, D_K), jnp.float32),
    scratch_types=(
        [pltpu.VMEM((R_K, D_K), jnp.float32) for _ in range(NB_K + 2)]
        + [pltpu.VMEM_SHARED((NS_K, NSLOT, H_K, D_K), jnp.float32)]
        + [pltpu.SemaphoreType.DMA for _ in range(NB_K + 2 + 2 * NSLOT + NDO)]
    ),
)
def _sc_add(x_hbm, pe_hbm, out_hbm, xb0, xb1, xb2, xb3, xb4, pb0, pb1, sp_stage,
            si0, si1, si2, si3, si4, sp0, sp1, sh10, sh11, sh20, sh21,
            sd0, sd1, sd2):
    xb = [xb0, xb1, xb2, xb3, xb4]
    si = [si0, si1, si2, si3, si4]
    pb = [pb0, pb1]
    sp = [sp0, sp1]
    sh1 = [sh10, sh11]
    sh2 = [sh20, sh21]
    sd = [sd0, sd1, sd2]

    cid = lax.axis_index("c")
    sid = lax.axis_index("s")
    wid = sid * NC_K + cid
    base = wid * ROWS_W

    def pe_row(c):
        return pl.multiple_of(base + c * R_K, R_K)

    def x_row(t):
        c, b = divmod(t, B_K)
        return pl.multiple_of(b * S_K + base + c * R_K, R_K)

    def in_copy(t):
        return pltpu.make_async_copy(
            x_hbm.at[pl.ds(x_row(t), R_K)], xb[t % NB_K], si[t % NB_K])

    def pe_copy(c):
        return pltpu.make_async_copy(
            pe_hbm.at[pl.ds(pe_row(c), R_K)], pb[c % 2], sp[c % 2])

    def hop1_copy(t):
        s = t % NSLOT
        return pltpu.make_async_copy(
            xb[t % NB_K].at[pl.ds(0, H_K)], sp_stage.at[sid, s], sh1[s])

    def hop2_copy(t):
        s = t % NSLOT
        return pltpu.make_async_copy(
            sp_stage.at[sid, s], out_hbm.at[pl.ds(x_row(t), H_K)], sh2[s])

    def dout_copy(t):
        return pltpu.make_async_copy(
            xb[t % NB_K].at[pl.ds(H_K, H_K)],
            out_hbm.at[pl.ds(x_row(t) + H_K, H_K)], sd[t % NDO])

    pe_copy(0).start()
    pe_copy(1).start()
    for t in range(LOOK_K):
        in_copy(t).start()

    for t in range(NT_K):
        nxt = t + LOOK_K
        if nxt < NT_K:
            if nxt >= NB_K:
                dout_copy(nxt - NB_K).wait()
            in_copy(nxt).start()

        c, b = divmod(t, B_K)
        if b == 0:
            pe_copy(c).wait()

        in_copy(t).wait()
        xbt = xb[t % NB_K]
        pebt = pb[c % 2]

        @plsc.parallel_loop(0, GROUPS_K, 1, unroll=16)
        def _add(g):
            r = g >> 6
            col = (g & (CPR_K - 1)) * L_K
            sl = pl.ds(col, L_K)
            xbt[r, sl] = xbt[r, sl] + pebt[r, sl] * SCALE_K

        if t >= NSLOT:
            hop2_copy(t - NSLOT).wait()
        hop1_copy(t).start()
        dout_copy(t).start()
        if t >= 1:
            hop1_copy(t - 1).wait()
            hop2_copy(t - 1).start()

        if b == B_K - 1 and c + 2 < CHUNKS_K:
            pe_copy(c + 2).start()

    hop1_copy(NT_K - 1).wait()
    hop2_copy(NT_K - 1).start()
    for t in range(NT_K - NSLOT, NT_K):
        hop2_copy(t).wait()
    for t in range(NT_K - NB_K + LOOK_K, NT_K):
        dout_copy(t).wait()


def kernel(x, pe_weight):
    b, s, d = x.shape
    out = _sc_add(x.reshape(b * s, d), pe_weight[:s])
    return out.reshape(b, s, d)
